# baseline (speedup 1.0000x reference)
import jax
import jax.numpy as jnp
from jax import lax
from jax.experimental import pallas as pl
from jax.experimental.pallas import tpu as pltpu
from jax.experimental.pallas import tpu_sc as plsc

C = 19
NB = 512
SUB = 2 * C * NB
HSZ = 2 * SUB
NW = 32
ROWS = 8
CHUNK = ROWS * 512
NCHUNK = 38 * 512 // ROWS // NW
NBUF = 4


def _bin_kernel(logits_ref, labels_ref, out_ref):
  x = logits_ref[0]
  e = jnp.exp(x)
  r = 1.0 / jnp.sum(e, axis=0, keepdims=True)
  p = e * r
  lbl = labels_ref[0]
  cls = lax.broadcasted_iota(jnp.int32, (C, 32, 512), 0)
  fg = lbl[None, :, :] == cls
  err = jnp.where(fg, 1.0 - p, p)
  b = (err * jnp.float32(NB - 0.001)).astype(jnp.int32)
  out_ref[...] = jnp.where(fg, C * NB, 0) + cls * NB + b


def _bin_indices(logits, labels, off):
  return pl.pallas_call(
      _bin_kernel,
      grid=(2, 16),
      in_specs=[
          pl.BlockSpec((1, C, 32, 512), lambda i, j: (i + off, 0, j, 0)),
          pl.BlockSpec((1, 32, 512), lambda i, j: (i + off, j, 0)),
      ],
      out_specs=pl.BlockSpec((C, 32, 512), lambda i, j: (i, j, 0)),
      out_shape=jax.ShapeDtypeStruct((2 * C, 512, 512), jnp.int32),
  )(logits, labels)


def _hist_body(idx_hbm, out_hbm, buf0, buf1, buf2, buf3, h0, h1,
               sem0, sem1, sem2, sem3):
  nc = 2
  wid = lax.axis_index("s") * nc + lax.axis_index("c")
  gbase = wid * NCHUNK

  zeros16 = jnp.zeros((16,), jnp.int32)

  def zero_body(i, carry):
    b = pl.multiple_of(i * 256, 256)
    for u in range(16):
      h0[pl.ds(b + u * 16, 16)] = zeros16
      h1[pl.ds(b + u * 16, 16)] = zeros16
    return carry
  lax.fori_loop(0, HSZ // 256, zero_body, 0)

  ones16 = jnp.ones((16,), jnp.int32)
  laneoff = (lax.iota(jnp.int32, 16) & 1) * SUB

  def start_copy(g, buf, sem):
    p = lax.shift_right_logical(g, 6)
    r = lax.mul(lax.bitwise_and(g, 63), ROWS)
    return pltpu.async_copy(idx_hbm.at[p, pl.ds(r, ROWS), :], buf, sem)

  def process(buf):
    lookahead = 8
    for u in range(ROWS):
      vs = [buf[u, pl.ds(k * 16, 16)] for k in range(lookahead)]
      for k in range(32):
        if k + lookahead < 32:
          vs.append(buf[u, pl.ds((k + lookahead) * 16, 16)])
        plsc.addupdate_scatter(h0 if k % 2 == 0 else h1,
                               [vs[k] + laneoff], ones16)

  bufs = [buf0, buf1, buf2, buf3]
  sems = [sem0, sem1, sem2, sem3]
  for s in range(NBUF - 1):
    start_copy(gbase + s, bufs[s], sems[s])

  def outer(t, carry):
    g0 = gbase + NBUF * t
    for s in range(NBUF):
      nxt = NBUF * t + s + (NBUF - 1)

      @pl.when(nxt < NCHUNK)
      def _start_next():
        start_copy(g0 + s + (NBUF - 1), bufs[(s + NBUF - 1) % NBUF],
                   sems[(s + NBUF - 1) % NBUF])

      pltpu.make_async_copy(idx_hbm.at[0, pl.ds(0, ROWS), :],
                            bufs[s], sems[s]).wait()
      process(bufs[s])
    return carry
  lax.fori_loop(0, NCHUNK // NBUF, outer, 0)

  pltpu.sync_copy(h0, out_hbm.at[2 * wid])
  pltpu.sync_copy(h1, out_hbm.at[2 * wid + 1])


def _histogram(idx):
  mesh = plsc.VectorSubcoreMesh(core_axis_name="c", subcore_axis_name="s")
  return pl.kernel(
      _hist_body,
      out_type=jax.ShapeDtypeStruct((2 * NW, HSZ), jnp.int32),
      mesh=mesh,
      compiler_params=pltpu.CompilerParams(needs_layout_passes=False),
      scratch_types=[
          pltpu.VMEM((ROWS, 512), jnp.int32),
          pltpu.VMEM((ROWS, 512), jnp.int32),
          pltpu.VMEM((ROWS, 512), jnp.int32),
          pltpu.VMEM((ROWS, 512), jnp.int32),
          pltpu.VMEM((HSZ,), jnp.int32),
          pltpu.VMEM((HSZ,), jnp.int32),
          pltpu.SemaphoreType.DMA,
          pltpu.SemaphoreType.DMA,
          pltpu.SemaphoreType.DMA,
          pltpu.SemaphoreType.DMA,
      ],
  )(idx)


def _final_kernel(p0_ref, p1_ref, out_ref):
  hs = jnp.sum(p0_ref[...], axis=0) + jnp.sum(p1_ref[...], axis=0)
  hs = (hs[:SUB] + hs[SUB:]).astype(jnp.float32)
  x = hs.reshape(2 * C, NB)
  s = 1
  while s < NB:
    x = x + jnp.concatenate(
        [x[:, s:], jnp.zeros((2 * C, s), jnp.float32)], axis=1)
    s *= 2
  ncum = x[:C] + x[C:]
  fcum = x[C:]
  g = fcum[:, 0:1]
  jac = 1.0 - (g - fcum) / jnp.maximum(g + ncum - fcum, 1.0)
  w = jnp.float32(1.0 / NB)
  loss_c = jnp.sum(jac, axis=1) * w - w * 0.5
  present = g[:, 0] > 0
  loss = (jnp.sum(jnp.where(present, loss_c, 0.0))
          / jnp.sum(present.astype(jnp.float32)))
  out_ref[...] = jnp.broadcast_to(loss, (1, 1))


def _finalize(parts0, parts1):
  return pl.pallas_call(
      _final_kernel,
      out_shape=jax.ShapeDtypeStruct((1, 1), jnp.float32),
  )(parts0, parts1)


@jax.jit
def kernel(logits, labels):
  idx0 = _bin_indices(logits, labels, 0)
  idx1 = _bin_indices(logits, labels, 2)
  parts0 = _histogram(idx0)
  parts1 = _histogram(idx1)
  return _finalize(parts0, parts1)[0, 0]


# speedup vs baseline: 1.0214x; 1.0214x over previous
import jax
import jax.numpy as jnp
from jax import lax
from jax.experimental import pallas as pl
from jax.experimental.pallas import tpu as pltpu
from jax.experimental.pallas import tpu_sc as plsc

C = 19
NB = 1024
SUB = 2 * C * NB
HSZ = SUB
NW = 32
ROWS = 4
CHUNK = ROWS * 512
NCHUNK = 38 * 512 // ROWS // NW
NBUF = 8


def _bin_kernel(logits_ref, labels_ref, out_ref):
  x = logits_ref[0]
  e = jnp.exp(x)
  r = 1.0 / jnp.sum(e, axis=0, keepdims=True)
  p = e * r
  lbl = labels_ref[0]
  cls = lax.broadcasted_iota(jnp.int32, (C, 32, 512), 0)
  fg = lbl[None, :, :] == cls
  err = jnp.where(fg, 1.0 - p, p)
  b = (err * jnp.float32(NB - 0.001)).astype(jnp.int32)
  out_ref[...] = jnp.where(fg, C * NB, 0) + cls * NB + b


def _bin_indices(logits, labels, off):
  return pl.pallas_call(
      _bin_kernel,
      grid=(2, 16),
      in_specs=[
          pl.BlockSpec((1, C, 32, 512), lambda i, j: (i + off, 0, j, 0)),
          pl.BlockSpec((1, 32, 512), lambda i, j: (i + off, j, 0)),
      ],
      out_specs=pl.BlockSpec((C, 32, 512), lambda i, j: (i, j, 0)),
      out_shape=jax.ShapeDtypeStruct((2 * C, 512, 512), jnp.int32),
  )(logits, labels)


def _hist_body(idx_hbm, out_hbm, buf0, buf1, buf2, buf3, buf4, buf5, buf6,
               buf7, h0, h1, sem0, sem1, sem2, sem3, sem4, sem5, sem6, sem7):
  nc = 2
  wid = lax.axis_index("s") * nc + lax.axis_index("c")
  gbase = wid * NCHUNK

  zeros16 = jnp.zeros((16,), jnp.int32)

  def zero_body(i, carry):
    b = pl.multiple_of(i * 256, 256)
    for u in range(16):
      h0[pl.ds(b + u * 16, 16)] = zeros16
      h1[pl.ds(b + u * 16, 16)] = zeros16
    return carry
  lax.fori_loop(0, HSZ // 256, zero_body, 0)

  ones16 = jnp.ones((16,), jnp.int32)

  def start_copy(g, buf, sem):
    p = lax.shift_right_logical(g, 7)
    r = lax.mul(lax.bitwise_and(g, 127), ROWS)
    return pltpu.async_copy(idx_hbm.at[p, pl.ds(r, ROWS), :], buf, sem)

  def process(buf):
    lookahead = 8
    for u in range(ROWS):
      vs = [buf[u, pl.ds(k * 16, 16)] for k in range(lookahead)]
      for k in range(32):
        if k + lookahead < 32:
          vs.append(buf[u, pl.ds((k + lookahead) * 16, 16)])
        plsc.addupdate_scatter(h0 if k % 2 == 0 else h1, [vs[k]], ones16)

  bufs = [buf0, buf1, buf2, buf3, buf4, buf5, buf6, buf7]
  sems = [sem0, sem1, sem2, sem3, sem4, sem5, sem6, sem7]
  for s in range(NBUF - 1):
    start_copy(gbase + s, bufs[s], sems[s])

  def outer(t, carry):
    g0 = gbase + NBUF * t
    for s in range(NBUF):
      nxt = NBUF * t + s + (NBUF - 1)

      @pl.when(nxt < NCHUNK)
      def _start_next():
        start_copy(g0 + s + (NBUF - 1), bufs[(s + NBUF - 1) % NBUF],
                   sems[(s + NBUF - 1) % NBUF])

      pltpu.make_async_copy(idx_hbm.at[0, pl.ds(0, ROWS), :],
                            bufs[s], sems[s]).wait()
      process(bufs[s])
    return carry
  lax.fori_loop(0, NCHUNK // NBUF, outer, 0)

  pltpu.sync_copy(h0, out_hbm.at[2 * wid])
  pltpu.sync_copy(h1, out_hbm.at[2 * wid + 1])


def _histogram(idx):
  mesh = plsc.VectorSubcoreMesh(core_axis_name="c", subcore_axis_name="s")
  return pl.kernel(
      _hist_body,
      out_type=jax.ShapeDtypeStruct((2 * NW, HSZ), jnp.int32),
      mesh=mesh,
      compiler_params=pltpu.CompilerParams(needs_layout_passes=False),
      scratch_types=(
          [pltpu.VMEM((ROWS, 512), jnp.int32)] * NBUF
          + [pltpu.VMEM((HSZ,), jnp.int32)] * 2
          + [pltpu.SemaphoreType.DMA] * NBUF
      ),
  )(idx)


def _final_kernel(p0_ref, p1_ref, out_ref):
  hs = (jnp.sum(p0_ref[...], axis=0)
        + jnp.sum(p1_ref[...], axis=0)).astype(jnp.float32)
  x = hs.reshape(2 * C, NB)
  s = 1
  while s < NB:
    x = x + jnp.concatenate(
        [x[:, s:], jnp.zeros((2 * C, s), jnp.float32)], axis=1)
    s *= 2
  ncum = x[:C] + x[C:]
  fcum = x[C:]
  g = fcum[:, 0:1]
  jac = 1.0 - (g - fcum) / jnp.maximum(g + ncum - fcum, 1.0)
  w = jnp.float32(1.0 / NB)
  loss_c = jnp.sum(jac, axis=1) * w - w * 0.5
  present = g[:, 0] > 0
  loss = (jnp.sum(jnp.where(present, loss_c, 0.0))
          / jnp.sum(present.astype(jnp.float32)))
  out_ref[...] = jnp.broadcast_to(loss, (1, 1))


def _finalize(parts0, parts1):
  return pl.pallas_call(
      _final_kernel,
      out_shape=jax.ShapeDtypeStruct((1, 1), jnp.float32),
  )(parts0, parts1)


@jax.jit
def kernel(logits, labels):
  idx0 = _bin_indices(logits, labels, 0)
  idx1 = _bin_indices(logits, labels, 2)
  parts0 = _histogram(idx0)
  parts1 = _histogram(idx1)
  return _finalize(parts0, parts1)[0, 0]
